# baseline (speedup 1.0000x reference)
import jax
import jax.numpy as jnp
from jax import lax
from jax.experimental import pallas as pl
from jax.experimental.pallas import tpu as pltpu
from jax.experimental.pallas import tpu_sc as plsc

N = 10000
E = 160000
M = 10000
TXT = 256
HID = 4096

NC = 2
NS = 16
NW = NC * NS
BK = 64
EW = 5184
E_PAD = NW * EW
NB = EW // BK
MP = 10240
CW = 128
ZR = MP // NS
GK = 3
ZB = 16

_F32 = jnp.float32
_MESH = dict(core_axis_name="c", subcore_axis_name="s",
             num_cores=NC, num_subcores=NS)


def _sc_pass(nch):
  out = jax.ShapeDtypeStruct((NC, nch, MP, CW), _F32)
  scratch = [
      pltpu.VMEM((NB, BK), jnp.int32),
      pltpu.VMEM((NB, BK), jnp.int32),
      pltpu.VMEM((GK * BK, CW), _F32),
      pltpu.VMEM((ZB, CW), _F32),
      pltpu.VMEM_SHARED((MP, CW), _F32),
      pltpu.SemaphoreType.DMA,
      pltpu.SemaphoreType.DMA,
      pltpu.SemaphoreType.DMA,
  ]

  def body(tbl, gidxm, sidxm, zrows, acc_out,
           gidx_t, sidx_t, rows_t, zbuf_t, acc_sh,
           gsem, ssem, zsem):
    ci = lax.axis_index("c")
    si = lax.axis_index("s")
    w = ci * NS + si
    pltpu.sync_copy(gidxm.at[w], gidx_t)
    pltpu.sync_copy(sidxm.at[w], sidx_t)
    pltpu.sync_copy(zrows, zbuf_t)

    def zero_own_slice():
      zd = [pltpu.async_copy(zbuf_t, acc_sh.at[pl.ds(si * ZR + z * ZB, ZB)],
                             zsem) for z in range(ZR // ZB)]
      for d in zd:
        d.wait()

    zero_own_slice()
    plsc.subcore_barrier()

    def chunk_body(c, carry):
      def grp_body(g, cc):
        gd = [pltpu.async_copy(tbl.at[c].at[gidx_t.at[g * GK + r]],
                               rows_t.at[pl.ds(r * BK, BK)], gsem)
              for r in range(GK)]
        for d in gd:
          d.wait()
        sd = [pltpu.async_copy(rows_t.at[pl.ds(r * BK, BK)],
                               acc_sh.at[sidx_t.at[g * GK + r]], ssem,
                               add=True)
              for r in range(GK)]
        for d in sd:
          d.wait()
        return cc
      lax.fori_loop(0, NB // GK, grp_body, 0)
      plsc.subcore_barrier()
      pltpu.sync_copy(acc_sh.at[pl.ds(si * ZR, ZR)],
                      acc_out.at[ci, c, pl.ds(si * ZR, ZR)])
      zero_own_slice()
      plsc.subcore_barrier()
      return carry
    lax.fori_loop(0, nch, chunk_body, 0)

  return pl.kernel(body, out_type=out,
                   mesh=plsc.VectorSubcoreMesh(**_MESH),
                   scratch_types=scratch)


_MB3 = 1024


def _scale_body(acc_ref, o_ref):
  m = pl.program_id(0)
  cnt = acc_ref[0, 2, :, 0:1] + acc_ref[1, 2, :, 0:1]
  binv = jnp.where(cnt > 0, 1.0 / cnt, 0.0)
  rows = m * _MB3 + lax.broadcasted_iota(jnp.int32, (_MB3, 1), 0)
  mask = rows < M
  for c in range(TXT // CW):
    o_ref[c] = jnp.where(mask, (acc_ref[0, c] + acc_ref[1, c]) * binv, 0.0)
  o_ref[TXT // CW] = jnp.where(
      mask, jnp.ones((_MB3, CW), _F32), jnp.zeros((_MB3, CW), _F32))


_MB5 = 512


def _mm_body(acc_ref, w1_ref, b1_ref, feat_ref, fch_ref):
  m = pl.program_id(0)
  cnt = acc_ref[0, 2, :, 0:1] + acc_ref[1, 2, :, 0:1]
  dinv = jnp.where(cnt > 0, 1.0 / cnt, 0.0)
  rows = m * _MB5 + lax.broadcasted_iota(jnp.int32, (_MB5, 1), 0)
  mask = rows < N
  xin = jnp.concatenate([acc_ref[0, 0] + acc_ref[1, 0],
                         acc_ref[0, 1] + acc_ref[1, 1]], axis=1)
  xin = jnp.where(mask, xin * dinv, 0.0)
  res = jnp.dot(xin, w1_ref[...], preferred_element_type=_F32) + b1_ref[...]
  res = jnp.where(mask, jnp.maximum(res, 0.0), 0.0)
  feat_ref[...] = res
  for c in range(HID // CW):
    fch_ref[c] = res[:, c * CW:(c + 1) * CW]


_MB7 = 1024


def _fin_body(acc_ref, cnt_ref, hid_ref, code_ref):
  cnt = cnt_ref[0, 0, :, 0:1] + cnt_ref[1, 0, :, 0:1]
  h = (acc_ref[0, 0] + acc_ref[1, 0]) / jnp.maximum(cnt, 1.0)
  hid_ref[...] = h
  code_ref[...] = jnp.tanh(h)


def kernel(x, hyperedge_index, W1, b1, W2, att2, b2):
  del W2, att2, b2
  idx0 = hyperedge_index[0]
  idx1 = hyperedge_index[1]
  pad = jnp.full((E_PAD - E,), N, jnp.int32)
  gid0 = jnp.concatenate([idx0, pad]).reshape(NW, NB, BK)
  gid1 = jnp.concatenate([idx1, pad]).reshape(NW, NB, BK)
  zrows = jnp.zeros((ZB, CW), _F32)

  xt = x.reshape(N, TXT // CW, CW).transpose(1, 0, 2)
  x_tbl = jnp.zeros((TXT // CW + 1, MP, CW), _F32).at[:TXT // CW, :N].set(xt)
  x_tbl = x_tbl.at[TXT // CW, :N].set(1.0)

  acc1 = _sc_pass(TXT // CW + 1)(x_tbl, gid0, gid1, zrows)

  z1 = pl.pallas_call(
      _scale_body,
      grid=(MP // _MB3,),
      in_specs=[
          pl.BlockSpec((NC, TXT // CW + 1, _MB3, CW), lambda m: (0, 0, m, 0)),
      ],
      out_specs=pl.BlockSpec((TXT // CW + 1, _MB3, CW), lambda m: (0, m, 0)),
      out_shape=jax.ShapeDtypeStruct((TXT // CW + 1, MP, CW), _F32),
  )(acc1)

  acc2 = _sc_pass(TXT // CW + 1)(z1, gid1, gid0, zrows)

  feat, f_tbl = pl.pallas_call(
      _mm_body,
      grid=(MP // _MB5,),
      in_specs=[
          pl.BlockSpec((NC, TXT // CW + 1, _MB5, CW), lambda m: (0, 0, m, 0)),
          pl.BlockSpec((TXT, HID), lambda m: (0, 0)),
          pl.BlockSpec((1, HID), lambda m: (0, 0)),
      ],
      out_specs=[
          pl.BlockSpec((_MB5, HID), lambda m: (m, 0)),
          pl.BlockSpec((HID // CW, _MB5, CW), lambda m: (0, m, 0)),
      ],
      out_shape=[
          jax.ShapeDtypeStruct((N, HID), _F32),
          jax.ShapeDtypeStruct((HID // CW, MP, CW), _F32),
      ],
  )(acc2, W1, b1.reshape(1, HID))

  acc3 = _sc_pass(HID // CW)(f_tbl, gid0, gid1, zrows)

  hid, code = pl.pallas_call(
      _fin_body,
      grid=(MP // _MB7, HID // CW),
      in_specs=[
          pl.BlockSpec((NC, 1, _MB7, CW), lambda m, c: (0, c, m, 0)),
          pl.BlockSpec((NC, 1, _MB7, CW), lambda m, c: (0, TXT // CW, m, 0)),
      ],
      out_specs=[
          pl.BlockSpec((_MB7, CW), lambda m, c: (m, c)),
          pl.BlockSpec((_MB7, CW), lambda m, c: (m, c)),
      ],
      out_shape=[
          jax.ShapeDtypeStruct((M, HID), _F32),
          jax.ShapeDtypeStruct((M, HID), _F32),
      ],
  )(acc3, acc1)

  return (feat, hid, code)


# speedup vs baseline: 1.4485x; 1.4485x over previous
import jax
import jax.numpy as jnp
from jax import lax
from jax.experimental import pallas as pl
from jax.experimental.pallas import tpu as pltpu
from jax.experimental.pallas import tpu_sc as plsc

N = 10000
E = 160000
M = 10000
TXT = 256
HID = 4096

NC = 2
NS = 16
NW = NC * NS
BK = 128
EW = 5120
E_PAD = NW * EW
NB = EW // BK
MP = 10240
CW = 128
ZR = MP // NS
ZB = 16

_F32 = jnp.float32
_MESH = dict(core_axis_name="c", subcore_axis_name="s",
             num_cores=NC, num_subcores=NS)


def _sc_pass(nch):
  out = jax.ShapeDtypeStruct((NC, nch, MP, CW), _F32)
  scratch = [
      pltpu.VMEM((NB, BK), jnp.int32),
      pltpu.VMEM((NB, BK), jnp.int32),
      pltpu.VMEM((2 * BK, CW), _F32),
      pltpu.VMEM((ZB, CW), _F32),
      pltpu.VMEM_SHARED((MP, CW), _F32),
      pltpu.SemaphoreType.DMA,
      pltpu.SemaphoreType.DMA,
  ]

  def body(tbl, gidxm, sidxm, zrows, acc_out,
           gidx_t, sidx_t, rows_t, zbuf_t, acc_sh, gsem, zsem):
    ci = lax.axis_index("c")
    si = lax.axis_index("s")
    w = ci * NS + si
    pltpu.sync_copy(gidxm.at[w], gidx_t)
    pltpu.sync_copy(sidxm.at[w], sidx_t)
    pltpu.sync_copy(zrows, zbuf_t)

    def zero_own_slice():
      zd = [pltpu.async_copy(zbuf_t, acc_sh.at[pl.ds(si * ZR + z * ZB, ZB)],
                             zsem) for z in range(ZR // ZB)]
      for d in zd:
        d.wait()

    zero_own_slice()
    plsc.subcore_barrier()

    def chunk_body(c, carry):
      tblc = tbl.at[c]

      def slot(b):
        return pl.ds(lax.rem(b, 2) * BK, BK)

      def issue_g(b):
        pltpu.async_copy(tblc.at[gidx_t.at[b]], rows_t.at[slot(b)], gsem)

      def drain_g():
        pltpu.make_async_copy(tblc.at[pl.ds(0, BK)],
                              rows_t.at[pl.ds(0, BK)], gsem).wait()

      issue_g(0)

      def b_body(b, cc):
        drain_g()
        @pl.when(b + 1 < NB)
        def _():
          issue_g(b + 1)
        pltpu.sync_copy(rows_t.at[slot(b)], acc_sh.at[sidx_t.at[b]],
                        add=True)
        return cc
      lax.fori_loop(0, NB, b_body, 0)
      plsc.subcore_barrier()
      pltpu.sync_copy(acc_sh.at[pl.ds(si * ZR, ZR)],
                      acc_out.at[ci, c, pl.ds(si * ZR, ZR)])
      zero_own_slice()
      plsc.subcore_barrier()
      return carry
    lax.fori_loop(0, nch, chunk_body, 0)

  return pl.kernel(body, out_type=out,
                   mesh=plsc.VectorSubcoreMesh(**_MESH),
                   scratch_types=scratch)


_MB3 = 1024


def _scale_body(acc_ref, o_ref):
  m = pl.program_id(0)
  cnt = acc_ref[0, 2, :, 0:1] + acc_ref[1, 2, :, 0:1]
  binv = jnp.where(cnt > 0, 1.0 / cnt, 0.0)
  rows = m * _MB3 + lax.broadcasted_iota(jnp.int32, (_MB3, 1), 0)
  mask = rows < M
  for c in range(TXT // CW):
    o_ref[c] = jnp.where(mask, (acc_ref[0, c] + acc_ref[1, c]) * binv, 0.0)
  o_ref[TXT // CW] = jnp.where(
      mask, jnp.ones((_MB3, CW), _F32), jnp.zeros((_MB3, CW), _F32))


_MB5 = 512


def _mm_body(acc_ref, w1_ref, b1_ref, feat_ref, fch_ref):
  m = pl.program_id(0)
  cnt = acc_ref[0, 2, :, 0:1] + acc_ref[1, 2, :, 0:1]
  dinv = jnp.where(cnt > 0, 1.0 / cnt, 0.0)
  rows = m * _MB5 + lax.broadcasted_iota(jnp.int32, (_MB5, 1), 0)
  mask = rows < N
  xin = jnp.concatenate([acc_ref[0, 0] + acc_ref[1, 0],
                         acc_ref[0, 1] + acc_ref[1, 1]], axis=1)
  xin = jnp.where(mask, xin * dinv, 0.0)
  res = jnp.dot(xin, w1_ref[...], preferred_element_type=_F32) + b1_ref[...]
  res = jnp.where(mask, jnp.maximum(res, 0.0), 0.0)
  feat_ref[...] = res
  for c in range(HID // CW):
    fch_ref[c] = res[:, c * CW:(c + 1) * CW]


_MB7 = 1024


def _fin_body(acc_ref, cnt_ref, hid_ref, code_ref):
  cnt = cnt_ref[0, 0, :, 0:1] + cnt_ref[1, 0, :, 0:1]
  h = (acc_ref[0, 0] + acc_ref[1, 0]) / jnp.maximum(cnt, 1.0)
  hid_ref[...] = h
  code_ref[...] = jnp.tanh(h)


def kernel(x, hyperedge_index, W1, b1, W2, att2, b2):
  del W2, att2, b2
  idx0 = hyperedge_index[0]
  idx1 = hyperedge_index[1]
  pad = jnp.full((E_PAD - E,), N, jnp.int32)
  gid0 = jnp.concatenate([idx0, pad]).reshape(NW, NB, BK)
  gid1 = jnp.concatenate([idx1, pad]).reshape(NW, NB, BK)
  zrows = jnp.zeros((ZB, CW), _F32)

  xt = x.reshape(N, TXT // CW, CW).transpose(1, 0, 2)
  x_tbl = jnp.zeros((TXT // CW + 1, MP, CW), _F32).at[:TXT // CW, :N].set(xt)
  x_tbl = x_tbl.at[TXT // CW, :N].set(1.0)

  acc1 = _sc_pass(TXT // CW + 1)(x_tbl, gid0, gid1, zrows)

  z1 = pl.pallas_call(
      _scale_body,
      grid=(MP // _MB3,),
      in_specs=[
          pl.BlockSpec((NC, TXT // CW + 1, _MB3, CW), lambda m: (0, 0, m, 0)),
      ],
      out_specs=pl.BlockSpec((TXT // CW + 1, _MB3, CW), lambda m: (0, m, 0)),
      out_shape=jax.ShapeDtypeStruct((TXT // CW + 1, MP, CW), _F32),
  )(acc1)

  acc2 = _sc_pass(TXT // CW + 1)(z1, gid1, gid0, zrows)

  feat, f_tbl = pl.pallas_call(
      _mm_body,
      grid=(MP // _MB5,),
      in_specs=[
          pl.BlockSpec((NC, TXT // CW + 1, _MB5, CW), lambda m: (0, 0, m, 0)),
          pl.BlockSpec((TXT, HID), lambda m: (0, 0)),
          pl.BlockSpec((1, HID), lambda m: (0, 0)),
      ],
      out_specs=[
          pl.BlockSpec((_MB5, HID), lambda m: (m, 0)),
          pl.BlockSpec((HID // CW, _MB5, CW), lambda m: (0, m, 0)),
      ],
      out_shape=[
          jax.ShapeDtypeStruct((N, HID), _F32),
          jax.ShapeDtypeStruct((HID // CW, MP, CW), _F32),
      ],
  )(acc2, W1, b1.reshape(1, HID))

  acc3 = _sc_pass(HID // CW)(f_tbl, gid0, gid1, zrows)

  hid, code = pl.pallas_call(
      _fin_body,
      grid=(MP // _MB7, HID // CW),
      in_specs=[
          pl.BlockSpec((NC, 1, _MB7, CW), lambda m, c: (0, c, m, 0)),
          pl.BlockSpec((NC, 1, _MB7, CW), lambda m, c: (0, TXT // CW, m, 0)),
      ],
      out_specs=[
          pl.BlockSpec((_MB7, CW), lambda m, c: (m, c)),
          pl.BlockSpec((_MB7, CW), lambda m, c: (m, c)),
      ],
      out_shape=[
          jax.ShapeDtypeStruct((M, HID), _F32),
          jax.ShapeDtypeStruct((M, HID), _F32),
      ],
  )(acc3, acc1)

  return (feat, hid, code)
